# baseline (speedup 1.0000x reference)
import functools

import jax
import jax.numpy as jnp
from jax import lax
from jax.experimental import pallas as pl
from jax.experimental.pallas import tpu as pltpu
from jax.experimental.pallas import tpu_sc as plsc

NN = 10000
EE = 320000
HF = 128
LEAK = 0.2

NC, NS, L = 2, 16, 16
NW = NC * NS
C = 112
NCHUNK = 90
EC = C * NCHUNK
EP = EC * NW
NP = 10008
NCHT = EP // C
NA = 72
NB = 180 - NA

BN = 1000
NBLK = NN // BN


def _tc_pre_body(x_ref, wt_ref, b_ref, av_ref, ab_ref, wh_ref, gr_ref):
    wh = jnp.dot(x_ref[...], wt_ref[...], preferred_element_type=jnp.float32)
    wh = wh + b_ref[...]
    wh_ref[...] = wh
    gr_ref[...] = (
        jnp.dot(wh, av_ref[...], preferred_element_type=jnp.float32)
        + ab_ref[...]
    )


def _tc_mid_body(acc_ref, s_ref, wt_ref, b_ref, av_ref, ab_ref,
                 wh_ref, gr_ref):
    s = s_ref[0] + s_ref[1] + 1e-16
    h = (acc_ref[0] + acc_ref[1]) / s
    h = jnp.where(h >= 0.0, h, LEAK * h)
    wh = jnp.dot(h, wt_ref[...], preferred_element_type=jnp.float32)
    wh = wh + b_ref[...]
    wh_ref[...] = wh
    gr_ref[...] = (
        jnp.dot(wh, av_ref[...], preferred_element_type=jnp.float32)
        + ab_ref[...]
    )


def _tc_post_body(acc_ref, s_ref, wt_ref, b_ref, out_ref):
    s = s_ref[0] + s_ref[1] + 1e-16
    h = (acc_ref[0] + acc_ref[1]) / s
    h = jnp.where(h >= 0.0, h, LEAK * h)
    out = jnp.dot(h, wt_ref[...], preferred_element_type=jnp.float32)
    out_ref[...] = out + b_ref[...]


_W_SPEC = pl.BlockSpec((HF, HF), lambda i: (0, 0))
_B_SPEC = pl.BlockSpec((1, HF), lambda i: (0, 0))

_tc_pre = pl.pallas_call(
    _tc_pre_body,
    grid=(NBLK,),
    in_specs=[pl.BlockSpec((BN, HF), lambda i: (i, 0)),
              _W_SPEC, _B_SPEC, _W_SPEC, _B_SPEC],
    out_specs=[pl.BlockSpec((BN, HF), lambda i: (i, 0)),
               pl.BlockSpec((BN, HF), lambda i: (i, 0))],
    out_shape=[jax.ShapeDtypeStruct((NN, HF), jnp.float32),
               jax.ShapeDtypeStruct((NN, HF), jnp.float32)],
)

_tc_mid = pl.pallas_call(
    _tc_mid_body,
    grid=(NBLK,),
    in_specs=[pl.BlockSpec((NC, BN, HF), lambda i: (0, i, 0)),
              pl.BlockSpec((NC, BN, 1), lambda i: (0, i, 0)),
              _W_SPEC, _B_SPEC, _W_SPEC, _B_SPEC],
    out_specs=[pl.BlockSpec((BN, HF), lambda i: (i, 0)),
               pl.BlockSpec((BN, HF), lambda i: (i, 0))],
    out_shape=[jax.ShapeDtypeStruct((NN, HF), jnp.float32),
               jax.ShapeDtypeStruct((NN, HF), jnp.float32)],
)

_tc_post = pl.pallas_call(
    _tc_post_body,
    grid=(NBLK,),
    in_specs=[pl.BlockSpec((NC, BN, HF), lambda i: (0, i, 0)),
              pl.BlockSpec((NC, BN, 1), lambda i: (0, i, 0)),
              _W_SPEC, _B_SPEC],
    out_specs=pl.BlockSpec((BN, HF), lambda i: (i, 0)),
    out_shape=jax.ShapeDtypeStruct((NN, HF), jnp.float32),
)


_sc_mesh = plsc.VectorSubcoreMesh(
    core_axis_name="c", subcore_axis_name="s", num_cores=NC, num_subcores=NS)


@functools.partial(
    pl.kernel,
    out_type=(jax.ShapeDtypeStruct((NC, NP, HF), jnp.float32),
              jax.ShapeDtypeStruct((NC, NP), jnp.float32)),
    mesh=_sc_mesh,
    scratch_types=[
        pltpu.VMEM((NP,), jnp.float32),
        pltpu.VMEM((NN,), jnp.float32),
        pltpu.VMEM((C,), jnp.int32),
        pltpu.VMEM((C,), jnp.int32),
        pltpu.VMEM((C,), jnp.int32),
        pltpu.VMEM((C,), jnp.int32),
        pltpu.VMEM((C,), jnp.int32),
        pltpu.VMEM((C,), jnp.int32),
        pltpu.VMEM((C, HF), jnp.float32),
        pltpu.VMEM((C, HF), jnp.float32),
        pltpu.VMEM((C,), jnp.float32),
        pltpu.VMEM_SHARED((NP, HF), jnp.float32),
        pltpu.VMEM_SHARED((NP,), jnp.float32),
        pltpu.SemaphoreType.DMA,
        pltpu.SemaphoreType.DMA,
        pltpu.SemaphoreType.DMA,
        pltpu.SemaphoreType.DMA,
        pltpu.SemaphoreType.DMA,
        pltpu.SemaphoreType.DMA,
    ],
    compiler_params=pltpu.CompilerParams(needs_layout_passes=False),
)
def _sc_edge(src_hbm, dst_hbm, g_hbm, r_hbm, wh_hbm, za_hbm, zs_hbm,
             acc_out, s_out, gv, rv, is0, is1, is2, id0, id1, id2,
             rows0, rows1, exb,
             acc, sacc, gsem0, gsem1, ssem, isem0, isem1, isem2):
    cid = lax.axis_index("c")
    sid = lax.axis_index("s")
    @pl.when(sid == 0)
    def _():
        pltpu.sync_copy(za_hbm, acc)
        pltpu.sync_copy(zs_hbm, sacc)

    pltpu.sync_copy(g_hbm, gv)
    pltpu.sync_copy(r_hbm, rv)
    plsc.subcore_barrier()

    isrc = (is0, is1, is2)
    idst = (id0, id1, id2)
    rows = (rows0, rows1)
    gsem = (gsem0, gsem1)
    isem = (isem0, isem1, isem2)

    my_n = NA + cid * (NB - NA)
    start = cid * (NS * NA) + sid * my_n

    def fetch_idx(cg, k):
        pltpu.async_copy(src_hbm.at[cg], isrc[k], isem[k])
        pltpu.async_copy(dst_hbm.at[cg], idst[k], isem[k])

    def wait_idx(cg, k):
        pltpu.make_async_copy(src_hbm.at[cg], isrc[k], isem[k]).wait()
        pltpu.make_async_copy(dst_hbm.at[cg], idst[k], isem[k]).wait()

    pltpu.sync_copy(src_hbm.at[start], is0)
    pltpu.sync_copy(dst_hbm.at[start], id0)
    fetch_idx(start + 1, 1)
    pltpu.async_copy(wh_hbm.at[is0], rows0, gsem0)

    def chunk6(i6, carry):
        for k in range(6):
            ci = i6 * 6 + k
            b = k % 2
            cur = k % 3
            nxt = (k + 1) % 3
            nn2 = (k + 2) % 3

            @pl.when(ci < my_n - 1)
            def _():
                wait_idx(start + ci + 1, nxt)
                pltpu.async_copy(wh_hbm.at[isrc[nxt]], rows[1 - b],
                                 gsem[1 - b])

            def grp(gi, c2):
                s16 = isrc[cur][pl.ds(gi * L, L)]
                d16 = idst[cur][pl.ds(gi * L, L)]
                t = plsc.load_gather(gv, [d16]) + plsc.load_gather(rv, [s16])
                t = jnp.where(t >= 0.0, t, LEAK * t)
                exb[pl.ds(gi * L, L)] = jnp.exp(t)
                return c2

            lax.fori_loop(0, C // L, grp, 0)

            pltpu.make_async_copy(wh_hbm.at[isrc[cur]], rows[b],
                                  gsem[b]).wait()

            def srow(j4, c2):
                for u in range(4):
                    j = j4 * 4 + u
                    es = plsc.load_gather(
                        exb, [jnp.full((L,), 0, jnp.int32) + j])
                    for q in range(HF // L):
                        rows[b][j, pl.ds(q * L, L)] = (
                            rows[b][j, pl.ds(q * L, L)] * es)
                return c2

            lax.fori_loop(0, C // 4, srow, 0)

            d1 = pltpu.async_copy(rows[b], acc.at[idst[cur]], ssem, add=True)
            d2 = pltpu.async_copy(exb, sacc.at[idst[cur]], ssem, add=True)

            @pl.when(ci < my_n - 2)
            def _():
                fetch_idx(start + ci + 2, nn2)

            d1.wait()
            d2.wait()
        return carry

    lax.fori_loop(0, my_n // 6, chunk6, 0)
    plsc.subcore_barrier()

    @pl.when(sid == 0)
    def _():
        pltpu.sync_copy(acc, acc_out.at[cid])
        pltpu.sync_copy(sacc, s_out.at[cid])


def _attn_vecs(a, ab):
    av = jnp.zeros((HF, HF), jnp.float32)
    av = av.at[:, 0].set(a[0, :HF])
    av = av.at[:, 1].set(a[0, HF:])
    abv = jnp.zeros((1, HF), jnp.float32).at[0, 0].set(ab[0])
    return av, abv


def kernel(x, edge_index, W1, b1, a1, ab1, W2, b2, a2, ab2, fcW, fcb):
    src = edge_index[0].astype(jnp.int32)
    dst = edge_index[1].astype(jnp.int32)
    pad = EP - EE
    src_p = jnp.concatenate([src, jnp.zeros((pad,), jnp.int32)])
    dst_p = jnp.concatenate([dst, jnp.full((pad,), NN, jnp.int32)])
    src_p = src_p.reshape(NCHT, C)
    dst_p = dst_p.reshape(NCHT, C)
    za = jnp.zeros((NP, HF), jnp.float32)
    zs = jnp.zeros((NP,), jnp.float32)

    av1, abv1 = _attn_vecs(a1, ab1)
    av2, abv2 = _attn_vecs(a2, ab2)

    wh1, gr1 = _tc_pre(x, W1.T, b1.reshape(1, HF), av1, abv1)
    g1 = jnp.pad(gr1[:, 0], (0, NP - NN)).at[NN].set(-1e30)
    r1 = gr1[:, 1]
    acc1, s1 = _sc_edge(src_p, dst_p, g1, r1, wh1, za, zs)

    wh2, gr2 = _tc_mid(acc1[:, :NN], s1[:, :NN].reshape(NC, NN, 1),
                       W2.T, b2.reshape(1, HF), av2, abv2)
    g2 = jnp.pad(gr2[:, 0], (0, NP - NN)).at[NN].set(-1e30)
    r2 = gr2[:, 1]
    acc2, s2 = _sc_edge(src_p, dst_p, g2, r2, wh2, za, zs)

    return _tc_post(acc2[:, :NN], s2[:, :NN].reshape(NC, NN, 1),
                    fcW.T, fcb.reshape(1, HF))


# speedup vs baseline: 1.1701x; 1.1701x over previous
import functools

import jax
import jax.numpy as jnp
from jax import lax
from jax.experimental import pallas as pl
from jax.experimental.pallas import tpu as pltpu
from jax.experimental.pallas import tpu_sc as plsc

NN = 10000
EE = 320000
HF = 128
LEAK = 0.2

NC, NS, L = 2, 16, 16
NW = NC * NS
C = 112
NCHUNK = 90
EC = C * NCHUNK
EP = EC * NW
NP = 10008
NCHT = EP // C
NA = 108
NB = 180 - NA

BN = 1000
NBLK = NN // BN


def _tc_pre_body(x_ref, wt_ref, b_ref, av_ref, ab_ref, wh_ref, gr_ref):
    wh = jnp.dot(x_ref[...], wt_ref[...], preferred_element_type=jnp.float32)
    wh = wh + b_ref[...]
    wh_ref[...] = wh
    gr_ref[...] = (
        jnp.dot(wh, av_ref[...], preferred_element_type=jnp.float32)
        + ab_ref[...]
    )


def _tc_mid_body(acc_ref, s_ref, wt_ref, b_ref, av_ref, ab_ref,
                 wh_ref, gr_ref):
    s = s_ref[0] + s_ref[1] + 1e-16
    h = (acc_ref[0] + acc_ref[1]) / s
    h = jnp.where(h >= 0.0, h, LEAK * h)
    wh = jnp.dot(h, wt_ref[...], preferred_element_type=jnp.float32)
    wh = wh + b_ref[...]
    wh_ref[...] = wh
    gr_ref[...] = (
        jnp.dot(wh, av_ref[...], preferred_element_type=jnp.float32)
        + ab_ref[...]
    )


def _tc_post_body(acc_ref, s_ref, wt_ref, b_ref, out_ref):
    s = s_ref[0] + s_ref[1] + 1e-16
    h = (acc_ref[0] + acc_ref[1]) / s
    h = jnp.where(h >= 0.0, h, LEAK * h)
    out = jnp.dot(h, wt_ref[...], preferred_element_type=jnp.float32)
    out_ref[...] = out + b_ref[...]


_W_SPEC = pl.BlockSpec((HF, HF), lambda i: (0, 0))
_B_SPEC = pl.BlockSpec((1, HF), lambda i: (0, 0))

_tc_pre = pl.pallas_call(
    _tc_pre_body,
    grid=(NBLK,),
    in_specs=[pl.BlockSpec((BN, HF), lambda i: (i, 0)),
              _W_SPEC, _B_SPEC, _W_SPEC, _B_SPEC],
    out_specs=[pl.BlockSpec((BN, HF), lambda i: (i, 0)),
               pl.BlockSpec((BN, HF), lambda i: (i, 0))],
    out_shape=[jax.ShapeDtypeStruct((NN, HF), jnp.float32),
               jax.ShapeDtypeStruct((NN, HF), jnp.float32)],
)

_tc_mid = pl.pallas_call(
    _tc_mid_body,
    grid=(NBLK,),
    in_specs=[pl.BlockSpec((NC, BN, HF), lambda i: (0, i, 0)),
              pl.BlockSpec((NC, BN, 1), lambda i: (0, i, 0)),
              _W_SPEC, _B_SPEC, _W_SPEC, _B_SPEC],
    out_specs=[pl.BlockSpec((BN, HF), lambda i: (i, 0)),
               pl.BlockSpec((BN, HF), lambda i: (i, 0))],
    out_shape=[jax.ShapeDtypeStruct((NN, HF), jnp.float32),
               jax.ShapeDtypeStruct((NN, HF), jnp.float32)],
)

_tc_post = pl.pallas_call(
    _tc_post_body,
    grid=(NBLK,),
    in_specs=[pl.BlockSpec((NC, BN, HF), lambda i: (0, i, 0)),
              pl.BlockSpec((NC, BN, 1), lambda i: (0, i, 0)),
              _W_SPEC, _B_SPEC],
    out_specs=pl.BlockSpec((BN, HF), lambda i: (i, 0)),
    out_shape=jax.ShapeDtypeStruct((NN, HF), jnp.float32),
)


_sc_mesh = plsc.VectorSubcoreMesh(
    core_axis_name="c", subcore_axis_name="s", num_cores=NC, num_subcores=NS)


@functools.partial(
    pl.kernel,
    out_type=(jax.ShapeDtypeStruct((NC, NP, HF), jnp.float32),
              jax.ShapeDtypeStruct((NC, NP), jnp.float32)),
    mesh=_sc_mesh,
    scratch_types=[
        pltpu.VMEM((NP,), jnp.float32),
        pltpu.VMEM((NN,), jnp.float32),
        pltpu.VMEM((C,), jnp.int32),
        pltpu.VMEM((C,), jnp.int32),
        pltpu.VMEM((C,), jnp.int32),
        pltpu.VMEM((C,), jnp.int32),
        pltpu.VMEM((C,), jnp.int32),
        pltpu.VMEM((C,), jnp.int32),
        pltpu.VMEM((C, HF), jnp.float32),
        pltpu.VMEM((C, HF), jnp.float32),
        pltpu.VMEM((C,), jnp.float32),
        pltpu.VMEM_SHARED((NP, HF), jnp.float32),
        pltpu.VMEM_SHARED((NP,), jnp.float32),
        pltpu.SemaphoreType.DMA,
        pltpu.SemaphoreType.DMA,
        pltpu.SemaphoreType.DMA,
        pltpu.SemaphoreType.DMA,
        pltpu.SemaphoreType.DMA,
        pltpu.SemaphoreType.DMA,
    ],
    compiler_params=pltpu.CompilerParams(needs_layout_passes=False),
)
def _sc_edge(src_hbm, dst_hbm, g_hbm, r_hbm, wh_hbm, za_hbm, zs_hbm,
             acc_out, s_out, gv, rv, is0, is1, is2, id0, id1, id2,
             rows0, rows1, exb,
             acc, sacc, gsem0, gsem1, ssem, isem0, isem1, isem2):
    cid = lax.axis_index("c")
    sid = lax.axis_index("s")
    @pl.when(sid == 0)
    def _():
        pltpu.sync_copy(za_hbm, acc)
        pltpu.sync_copy(zs_hbm, sacc)

    pltpu.sync_copy(g_hbm, gv)
    pltpu.sync_copy(r_hbm, rv)
    plsc.subcore_barrier()

    isrc = (is0, is1, is2)
    idst = (id0, id1, id2)
    rows = (rows0, rows1)
    gsem = (gsem0, gsem1)
    isem = (isem0, isem1, isem2)

    my_n = NA + cid * (NB - NA)
    start = cid * (NS * NA) + sid * my_n

    def fetch_idx(cg, k):
        pltpu.async_copy(src_hbm.at[cg], isrc[k], isem[k])
        pltpu.async_copy(dst_hbm.at[cg], idst[k], isem[k])

    def wait_idx(cg, k):
        pltpu.make_async_copy(src_hbm.at[cg], isrc[k], isem[k]).wait()
        pltpu.make_async_copy(dst_hbm.at[cg], idst[k], isem[k]).wait()

    pltpu.sync_copy(src_hbm.at[start], is0)
    pltpu.sync_copy(dst_hbm.at[start], id0)
    fetch_idx(start + 1, 1)
    pltpu.async_copy(wh_hbm.at[is0], rows0, gsem0)

    def chunk6(i6, carry):
        for k in range(6):
            ci = i6 * 6 + k
            b = k % 2
            cur = k % 3
            nxt = (k + 1) % 3
            nn2 = (k + 2) % 3

            @pl.when(ci < my_n - 1)
            def _():
                wait_idx(start + ci + 1, nxt)
                pltpu.async_copy(wh_hbm.at[isrc[nxt]], rows[1 - b],
                                 gsem[1 - b])

            def grp(gi, c2):
                s16 = isrc[cur][pl.ds(gi * L, L)]
                d16 = idst[cur][pl.ds(gi * L, L)]
                t = plsc.load_gather(gv, [d16]) + plsc.load_gather(rv, [s16])
                t = jnp.where(t >= 0.0, t, LEAK * t)
                exb[pl.ds(gi * L, L)] = jnp.exp(t)
                return c2

            lax.fori_loop(0, C // L, grp, 0)

            pltpu.make_async_copy(wh_hbm.at[isrc[cur]], rows[b],
                                  gsem[b]).wait()

            def srow(j4, c2):
                for u in range(4):
                    j = j4 * 4 + u
                    es = plsc.load_gather(
                        exb, [jnp.full((L,), 0, jnp.int32) + j])
                    for q in range(HF // L):
                        rows[b][j, pl.ds(q * L, L)] = (
                            rows[b][j, pl.ds(q * L, L)] * es)
                return c2

            lax.fori_loop(0, C // 4, srow, 0)

            d1 = pltpu.async_copy(rows[b], acc.at[idst[cur]], ssem, add=True)
            d2 = pltpu.async_copy(exb, sacc.at[idst[cur]], ssem, add=True)

            @pl.when(ci < my_n - 2)
            def _():
                fetch_idx(start + ci + 2, nn2)

            d1.wait()
            d2.wait()
        return carry

    lax.fori_loop(0, my_n // 6, chunk6, 0)
    plsc.subcore_barrier()

    @pl.when(sid == 0)
    def _():
        pltpu.sync_copy(acc, acc_out.at[cid])
        pltpu.sync_copy(sacc, s_out.at[cid])


def _attn_vecs(a, ab):
    av = jnp.zeros((HF, HF), jnp.float32)
    av = av.at[:, 0].set(a[0, :HF])
    av = av.at[:, 1].set(a[0, HF:])
    abv = jnp.zeros((1, HF), jnp.float32).at[0, 0].set(ab[0])
    return av, abv


def kernel(x, edge_index, W1, b1, a1, ab1, W2, b2, a2, ab2, fcW, fcb):
    src = edge_index[0].astype(jnp.int32)
    dst = edge_index[1].astype(jnp.int32)
    pad = EP - EE
    src_p = jnp.concatenate([src, jnp.zeros((pad,), jnp.int32)])
    dst_p = jnp.concatenate([dst, jnp.full((pad,), NN, jnp.int32)])
    src_p = src_p.reshape(NCHT, C)
    dst_p = dst_p.reshape(NCHT, C)
    za = jnp.zeros((NP, HF), jnp.float32)
    zs = jnp.zeros((NP,), jnp.float32)

    av1, abv1 = _attn_vecs(a1, ab1)
    av2, abv2 = _attn_vecs(a2, ab2)

    wh1, gr1 = _tc_pre(x, W1.T, b1.reshape(1, HF), av1, abv1)
    g1 = jnp.pad(gr1[:, 0], (0, NP - NN)).at[NN].set(-1e30)
    r1 = gr1[:, 1]
    acc1, s1 = _sc_edge(src_p, dst_p, g1, r1, wh1, za, zs)

    wh2, gr2 = _tc_mid(acc1[:, :NN], s1[:, :NN].reshape(NC, NN, 1),
                       W2.T, b2.reshape(1, HF), av2, abv2)
    g2 = jnp.pad(gr2[:, 0], (0, NP - NN)).at[NN].set(-1e30)
    r2 = gr2[:, 1]
    acc2, s2 = _sc_edge(src_p, dst_p, g2, r2, wh2, za, zs)

    return _tc_post(acc2[:, :NN], s2[:, :NN].reshape(NC, NN, 1),
                    fcW.T, fcb.reshape(1, HF))
